# Initial kernel scaffold; baseline (speedup 1.0000x reference)
import jax
import jax.numpy as jnp
from jax.experimental import pallas as pl


def kernel(hidden_states, ln1_w, Wq, Wk, Wv, q_norm_w, k_norm_w, Wo, ln2_w, router_W, W_gate, W_up, W_down):
    raise NotImplementedError("write your pallas kernel here")


# speedup vs baseline: 1.1014x; 1.1014x over previous
import functools
import jax
import jax.numpy as jnp
import numpy as np
from jax import lax
from jax.experimental import pallas as pl
from jax.experimental.pallas import tpu as pltpu

S, D = 2048, 2048
H, KVH, HD = 16, 4, 128
E, TOPK, F = 8, 2, 768
EPS = 1e-06
THETA = 10000.0
TR = 256
NT = (S * TOPK) // TR
NSEG = NT + E - 1
QT = 512
SB = 256


def _rms(x, w):
    return x * lax.rsqrt(jnp.mean(x * x, axis=-1, keepdims=True) + EPS) * w


def _qkv_body(x_ref, ln1_ref, qn_ref, kn_ref, w_ref, o_ref):
    s = pl.program_id(0)
    n = pl.program_id(1)
    x = x_ref[...]
    h = _rms(x, ln1_ref[...])
    y = jnp.dot(h, w_ref[...], preferred_element_type=jnp.float32)
    nh = y.shape[1] // HD
    y3 = y.reshape(SB, nh, HD)
    is_q = n < 4
    is_qk = n < 5
    wsel = jnp.where(is_q, qn_ref[...], kn_ref[...]).reshape(1, 1, HD)
    yn = y3 * lax.rsqrt(jnp.mean(y3 * y3, axis=-1, keepdims=True) + EPS) * wsel
    half = HD // 2
    pos = (lax.broadcasted_iota(jnp.int32, (SB, half), 0).astype(jnp.float32)
           + jnp.float32(s * SB))
    inv = 1.0 / (THETA ** (
        lax.broadcasted_iota(jnp.int32, (SB, half), 1).astype(jnp.float32)
        / half))
    f = pos * inv
    cos = jnp.cos(f)[:, None, :]
    sin = jnp.sin(f)[:, None, :]
    y1 = yn[..., :half]
    y2 = yn[..., half:]
    rot = jnp.concatenate([y1 * cos - y2 * sin, y2 * cos + y1 * sin], axis=-1)
    scale = jnp.where(is_q, jnp.float32(1.0 / np.sqrt(HD)), jnp.float32(1.0))
    out = jnp.where(is_qk, rot * scale, y3)
    o_ref[...] = out.reshape(SB, y.shape[1])


def _qkv_call(x, ln1, qn, kn, wqkv):
    nqkv = wqkv.shape[1]
    return pl.pallas_call(
        _qkv_body,
        grid=(S // SB, nqkv // 512),
        in_specs=[
            pl.BlockSpec((SB, D), lambda s, n: (s, 0)),
            pl.BlockSpec((1, D), lambda s, n: (0, 0)),
            pl.BlockSpec((1, HD), lambda s, n: (0, 0)),
            pl.BlockSpec((1, HD), lambda s, n: (0, 0)),
            pl.BlockSpec((D, 512), lambda s, n: (0, n)),
        ],
        out_specs=pl.BlockSpec((SB, 512), lambda s, n: (s, n)),
        out_shape=jax.ShapeDtypeStruct((S, nqkv), jnp.float32),
        compiler_params=pltpu.CompilerParams(
            dimension_semantics=("parallel", "arbitrary")),
    )(x, ln1, qn, kn, wqkv)


def _attn_body(q_ref, k_ref, v_ref, o_ref):
    t = pl.program_id(1)
    q = q_ref[...]
    k = k_ref[...]
    s = lax.dot_general(q, k, (((1,), (1,)), ((), ())),
                        preferred_element_type=jnp.float32)
    rowg = lax.broadcasted_iota(jnp.int32, (QT, S), 0) + t * QT
    colg = lax.broadcasted_iota(jnp.int32, (QT, S), 1)
    s = jnp.where(colg <= rowg, s, -1e30)
    m = jnp.max(s, axis=-1, keepdims=True)
    p = jnp.exp(s - m)
    a = p / jnp.sum(p, axis=-1, keepdims=True)
    o_ref[...] = jnp.dot(a, v_ref[...], preferred_element_type=jnp.float32)


def _attn_call(qkv):
    return pl.pallas_call(
        _attn_body,
        grid=(H, S // QT),
        in_specs=[
            pl.BlockSpec((QT, HD), lambda h, t: (t, h)),
            pl.BlockSpec((S, HD), lambda h, t: (0, H + h // (H // KVH))),
            pl.BlockSpec((S, HD), lambda h, t: (0, H + KVH + h // (H // KVH))),
        ],
        out_specs=pl.BlockSpec((QT, HD), lambda h, t: (t, h)),
        out_shape=jax.ShapeDtypeStruct((S, H * HD), jnp.float32),
        compiler_params=pltpu.CompilerParams(
            dimension_semantics=("arbitrary", "arbitrary")),
    )(qkv, qkv, qkv)


def _wo_body(o_ref, w_ref, r_ref, x_ref):
    x_ref[...] = r_ref[...] + jnp.dot(o_ref[...], w_ref[...],
                                      preferred_element_type=jnp.float32)


def _wo_call(o, wo, res):
    return pl.pallas_call(
        _wo_body,
        grid=(S // SB, 2),
        in_specs=[
            pl.BlockSpec((SB, D), lambda s, n: (s, 0)),
            pl.BlockSpec((D, D // 2), lambda s, n: (0, n)),
            pl.BlockSpec((SB, D // 2), lambda s, n: (s, n)),
        ],
        out_specs=pl.BlockSpec((SB, D // 2), lambda s, n: (s, n)),
        out_shape=jax.ShapeDtypeStruct((S, D), jnp.float32),
        compiler_params=pltpu.CompilerParams(
            dimension_semantics=("parallel", "arbitrary")),
    )(o, wo, res)


def _router_body(x_ref, ln2_ref, wp_ref, h2_ref, cw_ref, ii_ref):
    x = x_ref[...]
    h = _rms(x, ln2_ref[...])
    h2_ref[...] = h
    logits = jnp.dot(h, wp_ref[...], preferred_element_type=jnp.float32)
    col = lax.broadcasted_iota(jnp.int32, (SB, 128), 1)
    valid = col < E
    lm = jnp.where(valid, logits, -1e30)
    m = jnp.max(lm, axis=-1, keepdims=True)
    p = jnp.exp(lm - m)
    probs = p / jnp.sum(p, axis=-1, keepdims=True)
    m1 = jnp.max(probs, axis=-1, keepdims=True)
    i1 = jnp.min(jnp.where(probs == m1, col, 127), axis=-1, keepdims=True)
    probs2 = jnp.where(col == i1, -1.0, probs)
    m2 = jnp.max(probs2, axis=-1, keepdims=True)
    i2 = jnp.min(jnp.where(probs2 == m2, col, 127), axis=-1, keepdims=True)
    wsum = m1 + m2
    cw_ref[...] = (jnp.where(col == i1, m1 / wsum, 0.0)
                   + jnp.where(col == i2, m2 / wsum, 0.0))
    ii_ref[...] = jnp.where(col == 0, i1, jnp.where(col == 1, i2, 0))


def _router_call(x, ln2, wp):
    return pl.pallas_call(
        _router_body,
        grid=(S // SB,),
        in_specs=[
            pl.BlockSpec((SB, D), lambda s: (s, 0)),
            pl.BlockSpec((1, D), lambda s: (0, 0)),
            pl.BlockSpec((D, 128), lambda s: (0, 0)),
        ],
        out_specs=[
            pl.BlockSpec((SB, D), lambda s: (s, 0)),
            pl.BlockSpec((SB, 128), lambda s: (s, 0)),
            pl.BlockSpec((SB, 128), lambda s: (s, 0)),
        ],
        out_shape=[
            jax.ShapeDtypeStruct((S, D), jnp.float32),
            jax.ShapeDtypeStruct((S, 128), jnp.float32),
            jax.ShapeDtypeStruct((S, 128), jnp.int32),
        ],
        compiler_params=pltpu.CompilerParams(
            dimension_semantics=("parallel",)),
    )(x, ln2, wp)


def _gather_body(tok_ref, h_ref, o_ref):
    t = pl.program_id(0)

    def body(i, _):
        r = tok_ref[t * TR + i]
        o_ref[pl.ds(i, 1), :] = h_ref[pl.ds(r, 1), :]
        return 0

    lax.fori_loop(0, TR, body, 0)


def _gather_call(tok, h2):
    return pl.pallas_call(
        _gather_body,
        grid_spec=pltpu.PrefetchScalarGridSpec(
            num_scalar_prefetch=1,
            grid=(NT,),
            in_specs=[pl.BlockSpec((S, D), lambda t, tok: (0, 0))],
            out_specs=pl.BlockSpec((TR, D), lambda t, tok: (t, 0)),
        ),
        out_shape=jax.ShapeDtypeStruct((S * TOPK, D), jnp.float32),
    )(tok, h2)


def _grouped_body(tile_ref, eid_ref, s0_ref, s1_ref, first_ref,
                  hs_ref, wg_ref, wu_ref, wd_ref, ws_ref, o_ref):
    s = pl.program_id(0)
    x = hs_ref[...]
    wg = wg_ref[...].reshape(D, F)
    wu = wu_ref[...].reshape(D, F)
    wd = wd_ref[...].reshape(F, D)
    g = jnp.dot(x, wg, preferred_element_type=jnp.float32)
    u = jnp.dot(x, wu, preferred_element_type=jnp.float32)
    y = jnp.dot(g * lax.logistic(g) * u, wd,
                preferred_element_type=jnp.float32)
    rows = lax.broadcasted_iota(jnp.int32, (TR, 1), 0)
    mask = (rows >= s0_ref[s]) & (rows < s1_ref[s])
    contrib = jnp.where(mask, ws_ref[...] * y, 0.0)

    @pl.when(first_ref[s] == 1)
    def _():
        o_ref[...] = contrib

    @pl.when(first_ref[s] == 0)
    def _():
        o_ref[...] += contrib


def _grouped_call(tile_id, eid, s0, s1, first, hs, wg, wu, wd, ws_col):
    return pl.pallas_call(
        _grouped_body,
        grid_spec=pltpu.PrefetchScalarGridSpec(
            num_scalar_prefetch=5,
            grid=(NSEG,),
            in_specs=[
                pl.BlockSpec((TR, D), lambda s, t, e, a, b, f: (t[s], 0)),
                pl.BlockSpec((1, D, F), lambda s, t, e, a, b, f: (e[s], 0, 0)),
                pl.BlockSpec((1, D, F), lambda s, t, e, a, b, f: (e[s], 0, 0)),
                pl.BlockSpec((1, F, D), lambda s, t, e, a, b, f: (e[s], 0, 0)),
                pl.BlockSpec((TR, 1), lambda s, t, e, a, b, f: (t[s], 0)),
            ],
            out_specs=pl.BlockSpec((TR, D), lambda s, t, e, a, b, f: (t[s], 0)),
        ),
        out_shape=jax.ShapeDtypeStruct((S * TOPK, D), jnp.float32),
    )(tile_id, eid, s0, s1, first, hs, wg, wu, wd, ws_col)


def _combine_body(p1_ref, p2_ref, os_ref, r_ref, o_ref):
    t = pl.program_id(1)

    def body(i, _):
        a = os_ref[pl.ds(p1_ref[t * 128 + i], 1), :]
        b = os_ref[pl.ds(p2_ref[t * 128 + i], 1), :]
        o_ref[pl.ds(i, 1), :] = a + b + r_ref[pl.ds(i, 1), :]
        return 0

    lax.fori_loop(0, 128, body, 0)


def _combine_call(p1, p2, os_, res):
    return pl.pallas_call(
        _combine_body,
        grid_spec=pltpu.PrefetchScalarGridSpec(
            num_scalar_prefetch=2,
            grid=(4, S // 128),
            in_specs=[
                pl.BlockSpec((S * TOPK, D // 4), lambda c, t, p1, p2: (0, c)),
                pl.BlockSpec((128, D // 4), lambda c, t, p1, p2: (t, c)),
            ],
            out_specs=pl.BlockSpec((128, D // 4), lambda c, t, p1, p2: (t, c)),
        ),
        out_shape=jax.ShapeDtypeStruct((S, D), jnp.float32),
    )(p1, p2, os_, res)


def kernel(hidden_states, ln1_w, Wq, Wk, Wv, q_norm_w, k_norm_w, Wo, ln2_w,
           router_W, W_gate, W_up, W_down):
    x2 = hidden_states.reshape(S, D)
    wqkv = jnp.concatenate([Wq, Wk, Wv], axis=1)
    qkv = _qkv_call(x2, ln1_w.reshape(1, D), q_norm_w.reshape(1, HD),
                    k_norm_w.reshape(1, HD), wqkv)
    o = _attn_call(qkv)
    xo = _wo_call(o, Wo, x2)

    wp = jnp.pad(router_W, ((0, 0), (0, 128 - E)))
    h2, cw, ii = _router_call(xo, ln2_w.reshape(1, D), wp)

    topi = ii[:, :TOPK]
    w_flat = jnp.take_along_axis(cw[:, :E], topi, axis=1).reshape(-1)
    e_flat = topi.reshape(-1)
    order = jnp.argsort(e_flat, stable=True).astype(jnp.int32)
    tok = (order // TOPK).astype(jnp.int32)
    ws_col = w_flat[order][:, None]
    counts = jnp.sum(e_flat[None, :] == jnp.arange(E)[:, None], axis=1)
    csum = jnp.cumsum(counts)
    tile_starts = jnp.arange(NT + 1, dtype=jnp.int32) * TR
    bounds = jnp.sort(jnp.concatenate(
        [tile_starts, csum[:E - 1].astype(jnp.int32)]))
    seg0, seg1 = bounds[:NSEG], bounds[1:]
    tile_id = jnp.clip(seg0 // TR, 0, NT - 1).astype(jnp.int32)
    eid = jnp.clip(jnp.searchsorted(csum, seg0, side='right'),
                   0, E - 1).astype(jnp.int32)
    s0 = (seg0 - tile_id * TR).astype(jnp.int32)
    s1 = (seg1 - tile_id * TR).astype(jnp.int32)
    first = jnp.concatenate([jnp.ones((1,), jnp.int32),
                             (tile_id[1:] != tile_id[:-1]).astype(jnp.int32)])

    hs = _gather_call(tok, h2)
    os_ = _grouped_call(tile_id, eid, s0, s1, first, hs,
                        W_gate, W_up, W_down, ws_col)
    inv = jnp.argsort(order).astype(jnp.int32)
    y = _combine_call(inv[0::2], inv[1::2], os_, xo)
    return y.reshape(1, S, D)


# speedup vs baseline: 1.1667x; 1.0593x over previous
import functools
import jax
import jax.numpy as jnp
import numpy as np
from jax import lax
from jax.experimental import pallas as pl
from jax.experimental.pallas import tpu as pltpu
from jax.experimental.pallas import tpu_sc as plsc

S, D = 2048, 2048
H, KVH, HD = 16, 4, 128
E, TOPK, F = 8, 2, 768
EPS = 1e-06
THETA = 10000.0
TR = 256
NT = (S * TOPK) // TR
NSEG = NT + E - 1
QT = 512
SB = 256


def _rms(x, w):
    return x * lax.rsqrt(jnp.mean(x * x, axis=-1, keepdims=True) + EPS) * w


def _qkv_body(x_ref, ln1_ref, qn_ref, kn_ref, w_ref, o_ref):
    s = pl.program_id(0)
    n = pl.program_id(1)
    x = x_ref[...]
    h = _rms(x, ln1_ref[...])
    y = jnp.dot(h, w_ref[...], preferred_element_type=jnp.float32)
    nh = y.shape[1] // HD
    y3 = y.reshape(SB, nh, HD)
    is_q = n < 4
    is_qk = n < 5
    wsel = jnp.where(is_q, qn_ref[...], kn_ref[...]).reshape(1, 1, HD)
    yn = y3 * lax.rsqrt(jnp.mean(y3 * y3, axis=-1, keepdims=True) + EPS) * wsel
    half = HD // 2
    pos = (lax.broadcasted_iota(jnp.int32, (SB, half), 0).astype(jnp.float32)
           + jnp.float32(s * SB))
    inv = 1.0 / (THETA ** (
        lax.broadcasted_iota(jnp.int32, (SB, half), 1).astype(jnp.float32)
        / half))
    f = pos * inv
    cos = jnp.cos(f)[:, None, :]
    sin = jnp.sin(f)[:, None, :]
    y1 = yn[..., :half]
    y2 = yn[..., half:]
    rot = jnp.concatenate([y1 * cos - y2 * sin, y2 * cos + y1 * sin], axis=-1)
    scale = jnp.where(is_q, jnp.float32(1.0 / np.sqrt(HD)), jnp.float32(1.0))
    out = jnp.where(is_qk, rot * scale, y3)
    o_ref[...] = out.reshape(SB, y.shape[1])


def _qkv_call(x, ln1, qn, kn, wqkv):
    nqkv = wqkv.shape[1]
    return pl.pallas_call(
        _qkv_body,
        grid=(S // SB, nqkv // 512),
        in_specs=[
            pl.BlockSpec((SB, D), lambda s, n: (s, 0)),
            pl.BlockSpec((1, D), lambda s, n: (0, 0)),
            pl.BlockSpec((1, HD), lambda s, n: (0, 0)),
            pl.BlockSpec((1, HD), lambda s, n: (0, 0)),
            pl.BlockSpec((D, 512), lambda s, n: (0, n)),
        ],
        out_specs=pl.BlockSpec((SB, 512), lambda s, n: (s, n)),
        out_shape=jax.ShapeDtypeStruct((S, nqkv), jnp.float32),
        compiler_params=pltpu.CompilerParams(
            dimension_semantics=("parallel", "arbitrary")),
    )(x, ln1, qn, kn, wqkv)


def _attn_body(q_ref, k_ref, v_ref, o_ref):
    t = pl.program_id(1)
    q = q_ref[...]
    k = k_ref[...]
    s = lax.dot_general(q, k, (((1,), (1,)), ((), ())),
                        preferred_element_type=jnp.float32)
    rowg = lax.broadcasted_iota(jnp.int32, (QT, S), 0) + t * QT
    colg = lax.broadcasted_iota(jnp.int32, (QT, S), 1)
    s = jnp.where(colg <= rowg, s, -1e30)
    m = jnp.max(s, axis=-1, keepdims=True)
    p = jnp.exp(s - m)
    a = p / jnp.sum(p, axis=-1, keepdims=True)
    o_ref[...] = jnp.dot(a, v_ref[...], preferred_element_type=jnp.float32)


def _attn_call(qkv):
    return pl.pallas_call(
        _attn_body,
        grid=(H, S // QT),
        in_specs=[
            pl.BlockSpec((QT, HD), lambda h, t: (t, h)),
            pl.BlockSpec((S, HD), lambda h, t: (0, H + h // (H // KVH))),
            pl.BlockSpec((S, HD), lambda h, t: (0, H + KVH + h // (H // KVH))),
        ],
        out_specs=pl.BlockSpec((QT, HD), lambda h, t: (t, h)),
        out_shape=jax.ShapeDtypeStruct((S, H * HD), jnp.float32),
        compiler_params=pltpu.CompilerParams(
            dimension_semantics=("arbitrary", "arbitrary")),
    )(qkv, qkv, qkv)


def _wo_body(o_ref, w_ref, r_ref, x_ref):
    x_ref[...] = r_ref[...] + jnp.dot(o_ref[...], w_ref[...],
                                      preferred_element_type=jnp.float32)


def _wo_call(o, wo, res):
    return pl.pallas_call(
        _wo_body,
        grid=(S // SB, 2),
        in_specs=[
            pl.BlockSpec((SB, D), lambda s, n: (s, 0)),
            pl.BlockSpec((D, D // 2), lambda s, n: (0, n)),
            pl.BlockSpec((SB, D // 2), lambda s, n: (s, n)),
        ],
        out_specs=pl.BlockSpec((SB, D // 2), lambda s, n: (s, n)),
        out_shape=jax.ShapeDtypeStruct((S, D), jnp.float32),
        compiler_params=pltpu.CompilerParams(
            dimension_semantics=("parallel", "arbitrary")),
    )(o, wo, res)


def _router_body(x_ref, ln2_ref, wp_ref, h2_ref, cw_ref, ii_ref):
    x = x_ref[...]
    h = _rms(x, ln2_ref[...])
    h2_ref[...] = h
    logits = jnp.dot(h, wp_ref[...], preferred_element_type=jnp.float32)
    col = lax.broadcasted_iota(jnp.int32, (SB, 128), 1)
    valid = col < E
    lm = jnp.where(valid, logits, -1e30)
    m = jnp.max(lm, axis=-1, keepdims=True)
    p = jnp.exp(lm - m)
    probs = p / jnp.sum(p, axis=-1, keepdims=True)
    m1 = jnp.max(probs, axis=-1, keepdims=True)
    i1 = jnp.min(jnp.where(probs == m1, col, 127), axis=-1, keepdims=True)
    probs2 = jnp.where(col == i1, -1.0, probs)
    m2 = jnp.max(probs2, axis=-1, keepdims=True)
    i2 = jnp.min(jnp.where(probs2 == m2, col, 127), axis=-1, keepdims=True)
    wsum = m1 + m2
    cw_ref[...] = (jnp.where(col == i1, m1 / wsum, 0.0)
                   + jnp.where(col == i2, m2 / wsum, 0.0))
    ii_ref[...] = jnp.where(col == 0, i1, jnp.where(col == 1, i2, 0))


def _router_call(x, ln2, wp):
    return pl.pallas_call(
        _router_body,
        grid=(S // SB,),
        in_specs=[
            pl.BlockSpec((SB, D), lambda s: (s, 0)),
            pl.BlockSpec((1, D), lambda s: (0, 0)),
            pl.BlockSpec((D, 128), lambda s: (0, 0)),
        ],
        out_specs=[
            pl.BlockSpec((SB, D), lambda s: (s, 0)),
            pl.BlockSpec((SB, 128), lambda s: (s, 0)),
            pl.BlockSpec((SB, 128), lambda s: (s, 0)),
        ],
        out_shape=[
            jax.ShapeDtypeStruct((S, D), jnp.float32),
            jax.ShapeDtypeStruct((S, 128), jnp.float32),
            jax.ShapeDtypeStruct((S, 128), jnp.int32),
        ],
        compiler_params=pltpu.CompilerParams(
            dimension_semantics=("parallel",)),
    )(x, ln2, wp)


NW_SC = 32
ROWS_W = (S * TOPK) // NW_SC
CH = 16


def _sc_gather(table, idx):
    mesh = plsc.VectorSubcoreMesh(core_axis_name="c", subcore_axis_name="s")

    @functools.partial(
        pl.kernel, mesh=mesh,
        out_type=jax.ShapeDtypeStruct((S * TOPK, D), jnp.float32),
        scratch_types=[
            pltpu.VMEM((ROWS_W,), jnp.int32),
            pltpu.VMEM((CH, D), jnp.float32),
            pltpu.SemaphoreType.DMA,
        ],
    )
    def k(table_hbm, idx_hbm, out_hbm, idx_v, rows_v, sem):
        wid = lax.axis_index("s") * 2 + lax.axis_index("c")
        base = wid * ROWS_W
        pltpu.sync_copy(idx_hbm.at[pl.ds(base, ROWS_W)], idx_v)
        for c in range(ROWS_W // CH):
            pltpu.async_copy(
                table_hbm.at[idx_v.at[pl.ds(c * CH, CH)]], rows_v, sem).wait()
            pltpu.sync_copy(rows_v, out_hbm.at[pl.ds(base + c * CH, CH)])

    return k(table, idx)


def _grouped_body(tile_ref, eid_ref, s0_ref, s1_ref, first_ref,
                  hs_ref, wg_ref, wu_ref, wd_ref, ws_ref, o_ref):
    s = pl.program_id(0)
    x = hs_ref[...]
    wg = wg_ref[...].reshape(D, F)
    wu = wu_ref[...].reshape(D, F)
    wd = wd_ref[...].reshape(F, D)
    g = jnp.dot(x, wg, preferred_element_type=jnp.float32)
    u = jnp.dot(x, wu, preferred_element_type=jnp.float32)
    y = jnp.dot(g * lax.logistic(g) * u, wd,
                preferred_element_type=jnp.float32)
    rows = lax.broadcasted_iota(jnp.int32, (TR, 1), 0)
    mask = (rows >= s0_ref[s]) & (rows < s1_ref[s])
    contrib = jnp.where(mask, ws_ref[...] * y, 0.0)

    @pl.when(first_ref[s] == 1)
    def _():
        o_ref[...] = contrib

    @pl.when(first_ref[s] == 0)
    def _():
        o_ref[...] += contrib


def _grouped_call(tile_id, eid, s0, s1, first, hs, wg, wu, wd, ws_col):
    return pl.pallas_call(
        _grouped_body,
        grid_spec=pltpu.PrefetchScalarGridSpec(
            num_scalar_prefetch=5,
            grid=(NSEG,),
            in_specs=[
                pl.BlockSpec((TR, D), lambda s, t, e, a, b, f: (t[s], 0)),
                pl.BlockSpec((1, D, F), lambda s, t, e, a, b, f: (e[s], 0, 0)),
                pl.BlockSpec((1, D, F), lambda s, t, e, a, b, f: (e[s], 0, 0)),
                pl.BlockSpec((1, F, D), lambda s, t, e, a, b, f: (e[s], 0, 0)),
                pl.BlockSpec((TR, 1), lambda s, t, e, a, b, f: (t[s], 0)),
            ],
            out_specs=pl.BlockSpec((TR, D), lambda s, t, e, a, b, f: (t[s], 0)),
        ),
        out_shape=jax.ShapeDtypeStruct((S * TOPK, D), jnp.float32),
    )(tile_id, eid, s0, s1, first, hs, wg, wu, wd, ws_col)


def _fin_body(x_ref, ue_ref, uo_ref, o_ref):
    o_ref[...] = x_ref[...] + ue_ref[...] + uo_ref[...]


def _fin_call(xo, u2):
    return pl.pallas_call(
        _fin_body,
        grid=(S // SB,),
        in_specs=[
            pl.BlockSpec((SB, D), lambda s: (s, 0)),
            pl.BlockSpec((SB, D), lambda s: (s, 0)),
            pl.BlockSpec((SB, D), lambda s: (s, 1)),
        ],
        out_specs=pl.BlockSpec((SB, D), lambda s: (s, 0)),
        out_shape=jax.ShapeDtypeStruct((S, D), jnp.float32),
        compiler_params=pltpu.CompilerParams(
            dimension_semantics=("parallel",)),
    )(xo, u2, u2)


def kernel(hidden_states, ln1_w, Wq, Wk, Wv, q_norm_w, k_norm_w, Wo, ln2_w,
           router_W, W_gate, W_up, W_down):
    x2 = hidden_states.reshape(S, D)
    wqkv = jnp.concatenate([Wq, Wk, Wv], axis=1)
    qkv = _qkv_call(x2, ln1_w.reshape(1, D), q_norm_w.reshape(1, HD),
                    k_norm_w.reshape(1, HD), wqkv)
    o = _attn_call(qkv)
    xo = _wo_call(o, Wo, x2)

    wp = jnp.pad(router_W, ((0, 0), (0, 128 - E)))
    h2, cw, ii = _router_call(xo, ln2_w.reshape(1, D), wp)

    topi = ii[:, :TOPK]
    w_flat = jnp.take_along_axis(cw[:, :E], topi, axis=1).reshape(-1)
    e_flat = topi.reshape(-1)
    order = jnp.argsort(e_flat, stable=True).astype(jnp.int32)
    tok = (order // TOPK).astype(jnp.int32)
    ws_col = w_flat[order][:, None]
    counts = jnp.sum(e_flat[None, :] == jnp.arange(E)[:, None], axis=1)
    csum = jnp.cumsum(counts)
    tile_starts = jnp.arange(NT + 1, dtype=jnp.int32) * TR
    bounds = jnp.sort(jnp.concatenate(
        [tile_starts, csum[:E - 1].astype(jnp.int32)]))
    seg0, seg1 = bounds[:NSEG], bounds[1:]
    tile_id = jnp.clip(seg0 // TR, 0, NT - 1).astype(jnp.int32)
    eid = jnp.clip(jnp.searchsorted(csum, seg0, side='right'),
                   0, E - 1).astype(jnp.int32)
    s0 = (seg0 - tile_id * TR).astype(jnp.int32)
    s1 = (seg1 - tile_id * TR).astype(jnp.int32)
    first = jnp.concatenate([jnp.ones((1,), jnp.int32),
                             (tile_id[1:] != tile_id[:-1]).astype(jnp.int32)])

    hs = _sc_gather(h2, tok)
    os_ = _grouped_call(tile_id, eid, s0, s1, first, hs,
                        W_gate, W_up, W_down, ws_col)
    inv = jnp.argsort(order).astype(jnp.int32)
    u = _sc_gather(os_, inv)
    y = _fin_call(xo, u.reshape(S, 2 * D))
    return y.reshape(1, S, D)


# speedup vs baseline: 1.2189x; 1.0448x over previous
import functools
import jax
import jax.numpy as jnp
import numpy as np
from jax import lax
from jax.experimental import pallas as pl
from jax.experimental.pallas import tpu as pltpu
from jax.experimental.pallas import tpu_sc as plsc

S, D = 2048, 2048
H, KVH, HD = 16, 4, 128
E, TOPK, F = 8, 2, 768
EPS = 1e-06
THETA = 10000.0
TR = 256
NT = (S * TOPK) // TR
NSEG = NT + E - 1
QT = 512
SB = 256


def _rms(x, w):
    return x * lax.rsqrt(jnp.mean(x * x, axis=-1, keepdims=True) + EPS) * w


def _rope_norm(y3, w, pos_base):
    nh = y3.shape[1]
    half = HD // 2
    yn = (y3 * lax.rsqrt(jnp.mean(y3 * y3, axis=-1, keepdims=True) + EPS)
          * w.reshape(1, 1, HD))
    pos = (lax.broadcasted_iota(jnp.int32, (SB, half), 0).astype(jnp.float32)
           + jnp.float32(pos_base))
    inv = 1.0 / (THETA ** (
        lax.broadcasted_iota(jnp.int32, (SB, half), 1).astype(jnp.float32)
        / half))
    f = pos * inv
    cos = jnp.cos(f)[:, None, :]
    sin = jnp.sin(f)[:, None, :]
    y1 = yn[..., :half]
    y2 = yn[..., half:]
    return jnp.concatenate([y1 * cos - y2 * sin, y2 * cos + y1 * sin],
                           axis=-1).reshape(SB, nh * HD)


def _qkv_body(x_ref, ln1_ref, qn_ref, kn_ref, w_ref, o_ref):
    s = pl.program_id(0)
    x = x_ref[...]
    h = _rms(x, ln1_ref[...]).astype(jnp.bfloat16)
    y = jnp.dot(h, w_ref[...], preferred_element_type=jnp.float32)
    nq, nk = H * HD, KVH * HD
    qv = y[:, :nq].reshape(SB, H, HD)
    kv = y[:, nq:nq + nk].reshape(SB, KVH, HD)
    vv = y[:, nq + nk:]
    qr = _rope_norm(qv, qn_ref[...], s * SB) * jnp.float32(1.0 / np.sqrt(HD))
    kr = _rope_norm(kv, kn_ref[...], s * SB)
    o_ref[...] = jnp.concatenate([qr, kr, vv], axis=1).astype(jnp.bfloat16)


def _qkv_call(x, ln1, qn, kn, wqkv):
    nqkv = wqkv.shape[1]
    return pl.pallas_call(
        _qkv_body,
        grid=(S // SB,),
        in_specs=[
            pl.BlockSpec((SB, D), lambda s: (s, 0)),
            pl.BlockSpec((1, D), lambda s: (0, 0)),
            pl.BlockSpec((1, HD), lambda s: (0, 0)),
            pl.BlockSpec((1, HD), lambda s: (0, 0)),
            pl.BlockSpec((D, nqkv), lambda s: (0, 0)),
        ],
        out_specs=pl.BlockSpec((SB, nqkv), lambda s: (s, 0)),
        out_shape=jax.ShapeDtypeStruct((S, nqkv), jnp.bfloat16),
        compiler_params=pltpu.CompilerParams(
            dimension_semantics=("arbitrary",)),
    )(x, ln1, qn, kn, wqkv)


def _attn_body(q_ref, k_ref, v_ref, o_ref):
    t = pl.program_id(1)
    q = q_ref[...]
    k = k_ref[...]
    s = lax.dot_general(q, k, (((1,), (1,)), ((), ())),
                        preferred_element_type=jnp.float32)
    rowg = lax.broadcasted_iota(jnp.int32, (QT, S), 0) + t * QT
    colg = lax.broadcasted_iota(jnp.int32, (QT, S), 1)
    s = jnp.where(colg <= rowg, s, -1e30)
    m = jnp.max(s, axis=-1, keepdims=True)
    p = jnp.exp(s - m)
    a = (p / jnp.sum(p, axis=-1, keepdims=True)).astype(jnp.bfloat16)
    o_ref[...] = jnp.dot(a, v_ref[...],
                         preferred_element_type=jnp.float32).astype(jnp.bfloat16)


def _attn_call(qkv):
    return pl.pallas_call(
        _attn_body,
        grid=(H, S // QT),
        in_specs=[
            pl.BlockSpec((QT, HD), lambda h, t: (t, h)),
            pl.BlockSpec((S, HD), lambda h, t: (0, H + h // (H // KVH))),
            pl.BlockSpec((S, HD), lambda h, t: (0, H + KVH + h // (H // KVH))),
        ],
        out_specs=pl.BlockSpec((QT, HD), lambda h, t: (t, h)),
        out_shape=jax.ShapeDtypeStruct((S, H * HD), jnp.bfloat16),
        compiler_params=pltpu.CompilerParams(
            dimension_semantics=("arbitrary", "arbitrary")),
    )(qkv, qkv, qkv)


def _wo_body(o_ref, w_ref, r_ref, x_ref):
    x_ref[...] = r_ref[...] + jnp.dot(o_ref[...], w_ref[...],
                                      preferred_element_type=jnp.float32)


def _wo_call(o, wo, res):
    return pl.pallas_call(
        _wo_body,
        grid=(S // SB, 2),
        in_specs=[
            pl.BlockSpec((SB, D), lambda s, n: (s, 0)),
            pl.BlockSpec((D, D // 2), lambda s, n: (0, n)),
            pl.BlockSpec((SB, D // 2), lambda s, n: (s, n)),
        ],
        out_specs=pl.BlockSpec((SB, D // 2), lambda s, n: (s, n)),
        out_shape=jax.ShapeDtypeStruct((S, D), jnp.float32),
        compiler_params=pltpu.CompilerParams(
            dimension_semantics=("parallel", "arbitrary")),
    )(o, wo, res)


def _router_body(x_ref, ln2_ref, wp_ref, h2_ref, cw_ref, ii_ref):
    x = x_ref[...]
    h = _rms(x, ln2_ref[...])
    h2_ref[...] = h
    logits = jnp.dot(h, wp_ref[...], preferred_element_type=jnp.float32)
    col = lax.broadcasted_iota(jnp.int32, (SB, 128), 1)
    valid = col < E
    lm = jnp.where(valid, logits, -1e30)
    m = jnp.max(lm, axis=-1, keepdims=True)
    p = jnp.exp(lm - m)
    probs = p / jnp.sum(p, axis=-1, keepdims=True)
    m1 = jnp.max(probs, axis=-1, keepdims=True)
    i1 = jnp.min(jnp.where(probs == m1, col, 127), axis=-1, keepdims=True)
    probs2 = jnp.where(col == i1, -1.0, probs)
    m2 = jnp.max(probs2, axis=-1, keepdims=True)
    i2 = jnp.min(jnp.where(probs2 == m2, col, 127), axis=-1, keepdims=True)
    wsum = m1 + m2
    cw_ref[...] = (jnp.where(col == i1, m1 / wsum, 0.0)
                   + jnp.where(col == i2, m2 / wsum, 0.0))
    ii_ref[...] = jnp.where(col == 0, i1, jnp.where(col == 1, i2, 0))


def _router_call(x, ln2, wp):
    return pl.pallas_call(
        _router_body,
        grid=(S // SB,),
        in_specs=[
            pl.BlockSpec((SB, D), lambda s: (s, 0)),
            pl.BlockSpec((1, D), lambda s: (0, 0)),
            pl.BlockSpec((D, 128), lambda s: (0, 0)),
        ],
        out_specs=[
            pl.BlockSpec((SB, D), lambda s: (s, 0)),
            pl.BlockSpec((SB, 128), lambda s: (s, 0)),
            pl.BlockSpec((SB, 128), lambda s: (s, 0)),
        ],
        out_shape=[
            jax.ShapeDtypeStruct((S, D), jnp.float32),
            jax.ShapeDtypeStruct((S, 128), jnp.float32),
            jax.ShapeDtypeStruct((S, 128), jnp.int32),
        ],
        compiler_params=pltpu.CompilerParams(
            dimension_semantics=("parallel",)),
    )(x, ln2, wp)


NW_SC = 32
ROWS_W = (S * TOPK) // NW_SC
CH = 16


def _sc_gather(table, idx):
    mesh = plsc.VectorSubcoreMesh(core_axis_name="c", subcore_axis_name="s")

    @functools.partial(
        pl.kernel, mesh=mesh,
        out_type=jax.ShapeDtypeStruct((S * TOPK, D), jnp.float32),
        scratch_types=[
            pltpu.VMEM((ROWS_W,), jnp.int32),
            pltpu.VMEM((CH, D), jnp.float32),
            pltpu.SemaphoreType.DMA,
        ],
    )
    def k(table_hbm, idx_hbm, out_hbm, idx_v, rows_v, sem):
        wid = lax.axis_index("s") * 2 + lax.axis_index("c")
        base = wid * ROWS_W
        pltpu.sync_copy(idx_hbm.at[pl.ds(base, ROWS_W)], idx_v)
        for c in range(ROWS_W // CH):
            pltpu.async_copy(
                table_hbm.at[idx_v.at[pl.ds(c * CH, CH)]], rows_v, sem).wait()
            pltpu.sync_copy(rows_v, out_hbm.at[pl.ds(base + c * CH, CH)])

    return k(table, idx)


def _grouped_body(tile_ref, eid_ref, s0_ref, s1_ref, first_ref,
                  hs_ref, wg_ref, wu_ref, wd_ref, ws_ref, o_ref):
    s = pl.program_id(0)
    x = hs_ref[...].astype(jnp.bfloat16)
    wg = wg_ref[...].reshape(D, F)
    wu = wu_ref[...].reshape(D, F)
    wd = wd_ref[...].reshape(F, D)
    g = jnp.dot(x, wg, preferred_element_type=jnp.float32)
    u = jnp.dot(x, wu, preferred_element_type=jnp.float32)
    y = jnp.dot((g * lax.logistic(g) * u).astype(jnp.bfloat16), wd,
                preferred_element_type=jnp.float32)
    rows = lax.broadcasted_iota(jnp.int32, (TR, 1), 0)
    mask = (rows >= s0_ref[s]) & (rows < s1_ref[s])
    contrib = jnp.where(mask, ws_ref[...] * y, 0.0)

    @pl.when(first_ref[s] == 1)
    def _():
        o_ref[...] = contrib

    @pl.when(first_ref[s] == 0)
    def _():
        o_ref[...] += contrib


def _grouped_call(tile_id, eid, s0, s1, first, hs, wg, wu, wd, ws_col):
    return pl.pallas_call(
        _grouped_body,
        grid_spec=pltpu.PrefetchScalarGridSpec(
            num_scalar_prefetch=5,
            grid=(NSEG,),
            in_specs=[
                pl.BlockSpec((TR, D), lambda s, t, e, a, b, f: (t[s], 0)),
                pl.BlockSpec((1, D, F), lambda s, t, e, a, b, f: (e[s], 0, 0)),
                pl.BlockSpec((1, D, F), lambda s, t, e, a, b, f: (e[s], 0, 0)),
                pl.BlockSpec((1, F, D), lambda s, t, e, a, b, f: (e[s], 0, 0)),
                pl.BlockSpec((TR, 1), lambda s, t, e, a, b, f: (t[s], 0)),
            ],
            out_specs=pl.BlockSpec((TR, D), lambda s, t, e, a, b, f: (t[s], 0)),
        ),
        out_shape=jax.ShapeDtypeStruct((S * TOPK, D), jnp.float32),
    )(tile_id, eid, s0, s1, first, hs, wg, wu, wd, ws_col)


def _fin_body(x_ref, ue_ref, uo_ref, o_ref):
    o_ref[...] = x_ref[...] + ue_ref[...] + uo_ref[...]


def _fin_call(xo, u2):
    return pl.pallas_call(
        _fin_body,
        grid=(S // SB,),
        in_specs=[
            pl.BlockSpec((SB, D), lambda s: (s, 0)),
            pl.BlockSpec((SB, D), lambda s: (s, 0)),
            pl.BlockSpec((SB, D), lambda s: (s, 1)),
        ],
        out_specs=pl.BlockSpec((SB, D), lambda s: (s, 0)),
        out_shape=jax.ShapeDtypeStruct((S, D), jnp.float32),
        compiler_params=pltpu.CompilerParams(
            dimension_semantics=("parallel",)),
    )(xo, u2, u2)


def kernel(hidden_states, ln1_w, Wq, Wk, Wv, q_norm_w, k_norm_w, Wo, ln2_w,
           router_W, W_gate, W_up, W_down):
    x2 = hidden_states.reshape(S, D)
    wqkv = jnp.concatenate([Wq, Wk, Wv], axis=1).astype(jnp.bfloat16)
    qkv = _qkv_call(x2, ln1_w.reshape(1, D), q_norm_w.reshape(1, HD),
                    k_norm_w.reshape(1, HD), wqkv)
    o = _attn_call(qkv)
    xo = _wo_call(o, Wo.astype(jnp.bfloat16), x2)

    wp = jnp.pad(router_W, ((0, 0), (0, 128 - E)))
    h2, cw, ii = _router_call(xo, ln2_w.reshape(1, D), wp)

    topi = ii[:, :TOPK]
    w_flat = jnp.take_along_axis(cw[:, :E], topi, axis=1).reshape(-1)
    e_flat = topi.reshape(-1)
    order = jnp.argsort(e_flat, stable=True).astype(jnp.int32)
    tok = (order // TOPK).astype(jnp.int32)
    ws_col = w_flat[order][:, None]
    counts = jnp.sum(e_flat[None, :] == jnp.arange(E)[:, None], axis=1)
    csum = jnp.cumsum(counts)
    tile_starts = jnp.arange(NT + 1, dtype=jnp.int32) * TR
    bounds = jnp.sort(jnp.concatenate(
        [tile_starts, csum[:E - 1].astype(jnp.int32)]))
    seg0, seg1 = bounds[:NSEG], bounds[1:]
    tile_id = jnp.clip(seg0 // TR, 0, NT - 1).astype(jnp.int32)
    eid = jnp.clip(jnp.searchsorted(csum, seg0, side='right'),
                   0, E - 1).astype(jnp.int32)
    s0 = (seg0 - tile_id * TR).astype(jnp.int32)
    s1 = (seg1 - tile_id * TR).astype(jnp.int32)
    first = jnp.concatenate([jnp.ones((1,), jnp.int32),
                             (tile_id[1:] != tile_id[:-1]).astype(jnp.int32)])

    hs = _sc_gather(h2, tok)
    os_ = _grouped_call(tile_id, eid, s0, s1, first, hs,
                        W_gate.astype(jnp.bfloat16),
                        W_up.astype(jnp.bfloat16),
                        W_down.astype(jnp.bfloat16), ws_col)
    inv = jnp.argsort(order).astype(jnp.int32)
    u = _sc_gather(os_, inv)
    y = _fin_call(xo, u.reshape(S, 2 * D))
    return y.reshape(1, S, D)


# speedup vs baseline: 1.3641x; 1.1191x over previous
import functools
import jax
import jax.numpy as jnp
import numpy as np
from jax import lax
from jax.experimental import pallas as pl
from jax.experimental.pallas import tpu as pltpu
from jax.experimental.pallas import tpu_sc as plsc

S, D = 2048, 2048
H, KVH, HD = 16, 4, 128
E, TOPK, F = 8, 2, 768
EPS = 1e-06
THETA = 10000.0
TR = 256
NT = (S * TOPK) // TR
NSEG = NT + E - 1
QT = 512
SB = 256


def _rms(x, w):
    return x * lax.rsqrt(jnp.mean(x * x, axis=-1, keepdims=True) + EPS) * w


def _rope_norm(y3, w, pos_base):
    nh = y3.shape[1]
    half = HD // 2
    yn = (y3 * lax.rsqrt(jnp.mean(y3 * y3, axis=-1, keepdims=True) + EPS)
          * w.reshape(1, 1, HD))
    pos = (lax.broadcasted_iota(jnp.int32, (SB, half), 0).astype(jnp.float32)
           + jnp.float32(pos_base))
    inv = 1.0 / (THETA ** (
        lax.broadcasted_iota(jnp.int32, (SB, half), 1).astype(jnp.float32)
        / half))
    f = pos * inv
    cos = jnp.cos(f)[:, None, :]
    sin = jnp.sin(f)[:, None, :]
    y1 = yn[..., :half]
    y2 = yn[..., half:]
    return jnp.concatenate([y1 * cos - y2 * sin, y2 * cos + y1 * sin],
                           axis=-1).reshape(SB, nh * HD)


def _qkv_body(x_ref, ln1_ref, qn_ref, kn_ref, wq_ref, wk_ref, wv_ref, o_ref):
    s = pl.program_id(0)
    x = x_ref[...]
    h = _rms(x, ln1_ref[...]).astype(jnp.bfloat16)
    yq = jnp.dot(h, wq_ref[...].astype(jnp.bfloat16),
                 preferred_element_type=jnp.float32)
    yk = jnp.dot(h, wk_ref[...].astype(jnp.bfloat16),
                 preferred_element_type=jnp.float32)
    vv = jnp.dot(h, wv_ref[...].astype(jnp.bfloat16),
                 preferred_element_type=jnp.float32)
    qv = yq.reshape(SB, H, HD)
    kv = yk.reshape(SB, KVH, HD)
    qr = _rope_norm(qv, qn_ref[...], s * SB) * jnp.float32(1.0 / np.sqrt(HD))
    kr = _rope_norm(kv, kn_ref[...], s * SB)
    o_ref[...] = jnp.concatenate([qr, kr, vv], axis=1).astype(jnp.bfloat16)


def _qkv_call(x, ln1, qn, kn, wq, wk, wv):
    nqkv = (H + 2 * KVH) * HD
    return pl.pallas_call(
        _qkv_body,
        grid=(S // SB,),
        in_specs=[
            pl.BlockSpec((SB, D), lambda s: (s, 0)),
            pl.BlockSpec((1, D), lambda s: (0, 0)),
            pl.BlockSpec((1, HD), lambda s: (0, 0)),
            pl.BlockSpec((1, HD), lambda s: (0, 0)),
            pl.BlockSpec((D, H * HD), lambda s: (0, 0)),
            pl.BlockSpec((D, KVH * HD), lambda s: (0, 0)),
            pl.BlockSpec((D, KVH * HD), lambda s: (0, 0)),
        ],
        out_specs=pl.BlockSpec((SB, nqkv), lambda s: (s, 0)),
        out_shape=jax.ShapeDtypeStruct((S, nqkv), jnp.bfloat16),
        compiler_params=pltpu.CompilerParams(
            dimension_semantics=("arbitrary",)),
    )(x, ln1, qn, kn, wq, wk, wv)


def _attn_body(q_ref, k_ref, v_ref, o_ref):
    t = pl.program_id(1)
    q = q_ref[...]
    k = k_ref[...]
    s = lax.dot_general(q, k, (((1,), (1,)), ((), ())),
                        preferred_element_type=jnp.float32)
    rowg = lax.broadcasted_iota(jnp.int32, (QT, S), 0) + t * QT
    colg = lax.broadcasted_iota(jnp.int32, (QT, S), 1)
    s = jnp.where(colg <= rowg, s, -1e30)
    m = jnp.max(s, axis=-1, keepdims=True)
    p = jnp.exp(s - m)
    a = (p / jnp.sum(p, axis=-1, keepdims=True)).astype(jnp.bfloat16)
    o_ref[...] = jnp.dot(a, v_ref[...],
                         preferred_element_type=jnp.float32).astype(jnp.bfloat16)


def _attn_call(qkv):
    return pl.pallas_call(
        _attn_body,
        grid=(H, S // QT),
        in_specs=[
            pl.BlockSpec((QT, HD), lambda h, t: (t, h)),
            pl.BlockSpec((S, HD), lambda h, t: (0, H + h // (H // KVH))),
            pl.BlockSpec((S, HD), lambda h, t: (0, H + KVH + h // (H // KVH))),
        ],
        out_specs=pl.BlockSpec((QT, HD), lambda h, t: (t, h)),
        out_shape=jax.ShapeDtypeStruct((S, H * HD), jnp.bfloat16),
        compiler_params=pltpu.CompilerParams(
            dimension_semantics=("arbitrary", "arbitrary")),
    )(qkv, qkv, qkv)


def _wo_body(o_ref, w_ref, r_ref, x_ref):
    x_ref[...] = r_ref[...] + jnp.dot(o_ref[...],
                                      w_ref[...].astype(jnp.bfloat16),
                                      preferred_element_type=jnp.float32)


def _wo_call(o, wo, res):
    return pl.pallas_call(
        _wo_body,
        grid=(S // SB, 2),
        in_specs=[
            pl.BlockSpec((SB, D), lambda s, n: (s, 0)),
            pl.BlockSpec((D, D // 2), lambda s, n: (0, n)),
            pl.BlockSpec((SB, D // 2), lambda s, n: (s, n)),
        ],
        out_specs=pl.BlockSpec((SB, D // 2), lambda s, n: (s, n)),
        out_shape=jax.ShapeDtypeStruct((S, D), jnp.float32),
        compiler_params=pltpu.CompilerParams(
            dimension_semantics=("parallel", "arbitrary")),
    )(o, wo, res)


def _router_body(x_ref, ln2_ref, wp_ref, h2_ref, cw_ref, ii_ref):
    x = x_ref[...]
    h = _rms(x, ln2_ref[...])
    h2_ref[...] = h
    logits = jnp.dot(h, wp_ref[...], preferred_element_type=jnp.float32)
    col = lax.broadcasted_iota(jnp.int32, (SB, 128), 1)
    valid = col < E
    lm = jnp.where(valid, logits, -1e30)
    m = jnp.max(lm, axis=-1, keepdims=True)
    p = jnp.exp(lm - m)
    probs = p / jnp.sum(p, axis=-1, keepdims=True)
    m1 = jnp.max(probs, axis=-1, keepdims=True)
    i1 = jnp.min(jnp.where(probs == m1, col, 127), axis=-1, keepdims=True)
    probs2 = jnp.where(col == i1, -1.0, probs)
    m2 = jnp.max(probs2, axis=-1, keepdims=True)
    i2 = jnp.min(jnp.where(probs2 == m2, col, 127), axis=-1, keepdims=True)
    wsum = m1 + m2
    cw_ref[...] = (jnp.where(col == i1, m1 / wsum, 0.0)
                   + jnp.where(col == i2, m2 / wsum, 0.0))
    ii_ref[...] = jnp.where(col == 0, i1, jnp.where(col == 1, i2, 0))


def _router_call(x, ln2, wp):
    return pl.pallas_call(
        _router_body,
        grid=(S // SB,),
        in_specs=[
            pl.BlockSpec((SB, D), lambda s: (s, 0)),
            pl.BlockSpec((1, D), lambda s: (0, 0)),
            pl.BlockSpec((D, 128), lambda s: (0, 0)),
        ],
        out_specs=[
            pl.BlockSpec((SB, D), lambda s: (s, 0)),
            pl.BlockSpec((SB, 128), lambda s: (s, 0)),
            pl.BlockSpec((SB, 128), lambda s: (s, 0)),
        ],
        out_shape=[
            jax.ShapeDtypeStruct((S, D), jnp.float32),
            jax.ShapeDtypeStruct((S, 128), jnp.float32),
            jax.ShapeDtypeStruct((S, 128), jnp.int32),
        ],
        compiler_params=pltpu.CompilerParams(
            dimension_semantics=("parallel",)),
    )(x, ln2, wp)


NW_SC = 32
ROWS_W = (S * TOPK) // NW_SC
CH = 16


def _sc_gather(table, idx):
    mesh = plsc.VectorSubcoreMesh(core_axis_name="c", subcore_axis_name="s")

    nch = ROWS_W // CH

    @functools.partial(
        pl.kernel, mesh=mesh,
        out_type=jax.ShapeDtypeStruct((S * TOPK, D), jnp.float32),
        scratch_types=[
            pltpu.VMEM((ROWS_W,), jnp.int32),
            pltpu.VMEM((CH, D), jnp.float32),
            pltpu.VMEM((CH, D), jnp.float32),
            pltpu.SemaphoreType.DMA,
            pltpu.SemaphoreType.DMA,
        ],
    )
    def k(table_hbm, idx_hbm, out_hbm, idx_v, rows0, rows1, sem0, sem1):
        wid = lax.axis_index("s") * 2 + lax.axis_index("c")
        base = wid * ROWS_W
        pltpu.sync_copy(idx_hbm.at[pl.ds(base, ROWS_W)], idx_v)
        bufs = (rows0, rows1)
        sems = (sem0, sem1)
        handles = [None] * nch
        for c in range(2):
            handles[c] = pltpu.async_copy(
                table_hbm.at[idx_v.at[pl.ds(c * CH, CH)]], bufs[c], sems[c])
        for c in range(nch):
            handles[c].wait()
            pltpu.sync_copy(bufs[c % 2], out_hbm.at[pl.ds(base + c * CH, CH)])
            if c + 2 < nch:
                handles[c + 2] = pltpu.async_copy(
                    table_hbm.at[idx_v.at[pl.ds((c + 2) * CH, CH)]],
                    bufs[c % 2], sems[c % 2])

    return k(table, idx)


def _grouped_body(tile_ref, eid_ref, s0_ref, s1_ref, first_ref,
                  hs_ref, wg_ref, wu_ref, wd_ref, ws_ref, o_ref):
    s = pl.program_id(0)
    x = hs_ref[...].astype(jnp.bfloat16)
    wg = wg_ref[...].reshape(D, F).astype(jnp.bfloat16)
    wu = wu_ref[...].reshape(D, F).astype(jnp.bfloat16)
    wd = wd_ref[...].reshape(F, D).astype(jnp.bfloat16)
    g = jnp.dot(x, wg, preferred_element_type=jnp.float32)
    u = jnp.dot(x, wu, preferred_element_type=jnp.float32)
    y = jnp.dot((g * lax.logistic(g) * u).astype(jnp.bfloat16), wd,
                preferred_element_type=jnp.float32)
    rows = lax.broadcasted_iota(jnp.int32, (TR, 1), 0)
    mask = (rows >= s0_ref[s]) & (rows < s1_ref[s])
    contrib = jnp.where(mask, ws_ref[...] * y, 0.0)

    @pl.when(first_ref[s] == 1)
    def _():
        o_ref[...] = contrib

    @pl.when(first_ref[s] == 0)
    def _():
        o_ref[...] += contrib


def _grouped_call(tile_id, eid, s0, s1, first, hs, wg, wu, wd, ws_col):
    return pl.pallas_call(
        _grouped_body,
        grid_spec=pltpu.PrefetchScalarGridSpec(
            num_scalar_prefetch=5,
            grid=(NSEG,),
            in_specs=[
                pl.BlockSpec((TR, D), lambda s, t, e, a, b, f: (t[s], 0)),
                pl.BlockSpec((1, D, F), lambda s, t, e, a, b, f: (e[s], 0, 0)),
                pl.BlockSpec((1, D, F), lambda s, t, e, a, b, f: (e[s], 0, 0)),
                pl.BlockSpec((1, F, D), lambda s, t, e, a, b, f: (e[s], 0, 0)),
                pl.BlockSpec((TR, 1), lambda s, t, e, a, b, f: (t[s], 0)),
            ],
            out_specs=pl.BlockSpec((TR, D), lambda s, t, e, a, b, f: (t[s], 0)),
        ),
        out_shape=jax.ShapeDtypeStruct((S * TOPK, D), jnp.float32),
    )(tile_id, eid, s0, s1, first, hs, wg, wu, wd, ws_col)


def _fin_body(x_ref, ue_ref, uo_ref, o_ref):
    o_ref[...] = x_ref[...] + ue_ref[...] + uo_ref[...]


def _fin_call(xo, u2):
    return pl.pallas_call(
        _fin_body,
        grid=(S // SB,),
        in_specs=[
            pl.BlockSpec((SB, D), lambda s: (s, 0)),
            pl.BlockSpec((SB, D), lambda s: (s, 0)),
            pl.BlockSpec((SB, D), lambda s: (s, 1)),
        ],
        out_specs=pl.BlockSpec((SB, D), lambda s: (s, 0)),
        out_shape=jax.ShapeDtypeStruct((S, D), jnp.float32),
        compiler_params=pltpu.CompilerParams(
            dimension_semantics=("parallel",)),
    )(xo, u2, u2)


def kernel(hidden_states, ln1_w, Wq, Wk, Wv, q_norm_w, k_norm_w, Wo, ln2_w,
           router_W, W_gate, W_up, W_down):
    x2 = hidden_states.reshape(S, D)
    qkv = _qkv_call(x2, ln1_w.reshape(1, D), q_norm_w.reshape(1, HD),
                    k_norm_w.reshape(1, HD), Wq, Wk, Wv)
    o = _attn_call(qkv)
    xo = _wo_call(o, Wo, x2)

    wp = jnp.pad(router_W, ((0, 0), (0, 128 - E)))
    h2, cw, ii = _router_call(xo, ln2_w.reshape(1, D), wp)

    topi = ii[:, :TOPK]
    w_flat = jnp.take_along_axis(cw[:, :E], topi, axis=1).reshape(-1)
    e_flat = topi.reshape(-1)
    order = jnp.argsort(e_flat, stable=True).astype(jnp.int32)
    tok = (order // TOPK).astype(jnp.int32)
    ws_col = w_flat[order][:, None]
    counts = jnp.sum(e_flat[None, :] == jnp.arange(E)[:, None], axis=1)
    csum = jnp.cumsum(counts)
    tile_starts = jnp.arange(NT + 1, dtype=jnp.int32) * TR
    bounds = jnp.sort(jnp.concatenate(
        [tile_starts, csum[:E - 1].astype(jnp.int32)]))
    seg0, seg1 = bounds[:NSEG], bounds[1:]
    tile_id = jnp.clip(seg0 // TR, 0, NT - 1).astype(jnp.int32)
    eid = jnp.clip(jnp.searchsorted(csum, seg0, side='right'),
                   0, E - 1).astype(jnp.int32)
    s0 = (seg0 - tile_id * TR).astype(jnp.int32)
    s1 = (seg1 - tile_id * TR).astype(jnp.int32)
    first = jnp.concatenate([jnp.ones((1,), jnp.int32),
                             (tile_id[1:] != tile_id[:-1]).astype(jnp.int32)])

    hs = _sc_gather(h2, tok)
    os_ = _grouped_call(tile_id, eid, s0, s1, first, hs,
                        W_gate, W_up, W_down, ws_col)
    inv = jnp.argsort(order).astype(jnp.int32)
    u = _sc_gather(os_, inv)
    y = _fin_call(xo, u.reshape(S, 2 * D))
    return y.reshape(1, S, D)


# speedup vs baseline: 1.6398x; 1.2021x over previous
import functools
import jax
import jax.numpy as jnp
import numpy as np
from jax import lax
from jax.experimental import pallas as pl
from jax.experimental.pallas import tpu as pltpu
from jax.experimental.pallas import tpu_sc as plsc

S, D = 2048, 2048
H, KVH, HD = 16, 4, 128
E, TOPK, F = 8, 2, 768
EPS = 1e-06
THETA = 10000.0
TR = 256
NT = (S * TOPK) // TR
NSEG = NT + E - 1
QT = 512
SB = 256


def _rms(x, w):
    return x * lax.rsqrt(jnp.mean(x * x, axis=-1, keepdims=True) + EPS) * w


def _rope_norm(y3, w, pos_base):
    nh = y3.shape[1]
    half = HD // 2
    yn = (y3 * lax.rsqrt(jnp.mean(y3 * y3, axis=-1, keepdims=True) + EPS)
          * w.reshape(1, 1, HD))
    pos = (lax.broadcasted_iota(jnp.int32, (SB, half), 0).astype(jnp.float32)
           + jnp.float32(pos_base))
    inv = 1.0 / (THETA ** (
        lax.broadcasted_iota(jnp.int32, (SB, half), 1).astype(jnp.float32)
        / half))
    f = pos * inv
    cos = jnp.cos(f)[:, None, :]
    sin = jnp.sin(f)[:, None, :]
    y1 = yn[..., :half]
    y2 = yn[..., half:]
    return jnp.concatenate([y1 * cos - y2 * sin, y2 * cos + y1 * sin],
                           axis=-1).reshape(SB, nh * HD)


def _qkv_body(x_ref, ln1_ref, qn_ref, kn_ref, wq_ref, wk_ref, wv_ref, o_ref):
    s = pl.program_id(0)
    x = x_ref[...]
    h = _rms(x, ln1_ref[...]).astype(jnp.bfloat16)
    yq = jnp.dot(h, wq_ref[...].astype(jnp.bfloat16),
                 preferred_element_type=jnp.float32)
    yk = jnp.dot(h, wk_ref[...].astype(jnp.bfloat16),
                 preferred_element_type=jnp.float32)
    vv = jnp.dot(h, wv_ref[...].astype(jnp.bfloat16),
                 preferred_element_type=jnp.float32)
    qv = yq.reshape(SB, H, HD)
    kv = yk.reshape(SB, KVH, HD)
    qr = _rope_norm(qv, qn_ref[...], s * SB) * jnp.float32(1.0 / np.sqrt(HD))
    kr = _rope_norm(kv, kn_ref[...], s * SB)
    o_ref[...] = jnp.concatenate([qr, kr, vv], axis=1).astype(jnp.bfloat16)


def _qkv_call(x, ln1, qn, kn, wq, wk, wv):
    nqkv = (H + 2 * KVH) * HD
    return pl.pallas_call(
        _qkv_body,
        grid=(S // SB,),
        in_specs=[
            pl.BlockSpec((SB, D), lambda s: (s, 0)),
            pl.BlockSpec((1, D), lambda s: (0, 0)),
            pl.BlockSpec((1, HD), lambda s: (0, 0)),
            pl.BlockSpec((1, HD), lambda s: (0, 0)),
            pl.BlockSpec((D, H * HD), lambda s: (0, 0)),
            pl.BlockSpec((D, KVH * HD), lambda s: (0, 0)),
            pl.BlockSpec((D, KVH * HD), lambda s: (0, 0)),
        ],
        out_specs=pl.BlockSpec((SB, nqkv), lambda s: (s, 0)),
        out_shape=jax.ShapeDtypeStruct((S, nqkv), jnp.bfloat16),
        compiler_params=pltpu.CompilerParams(
            dimension_semantics=("arbitrary",)),
    )(x, ln1, qn, kn, wq, wk, wv)


def _attn_body(q_ref, k_ref, v_ref, o_ref):
    t = pl.program_id(1)
    q = q_ref[...]
    rowg = lax.broadcasted_iota(jnp.int32, (QT, QT), 0) + t * QT
    colg = lax.broadcasted_iota(jnp.int32, (QT, QT), 1)

    def step(j, carry):
        m, l, acc = carry
        kb = k_ref[pl.ds(j * QT, QT), :]
        s = lax.dot_general(q, kb, (((1,), (1,)), ((), ())),
                            preferred_element_type=jnp.float32)
        s = jnp.where(colg + j * QT <= rowg, s, -1e30)
        mn = jnp.maximum(m, jnp.max(s, axis=-1, keepdims=True))
        p = jnp.exp(s - mn)
        corr = jnp.exp(m - mn)
        l2 = l * corr + jnp.sum(p, axis=-1, keepdims=True)
        vb = v_ref[pl.ds(j * QT, QT), :]
        acc2 = acc * corr + jnp.dot(p.astype(jnp.bfloat16), vb,
                                    preferred_element_type=jnp.float32)
        return mn, l2, acc2

    m0 = jnp.full((QT, 1), -1e30, jnp.float32)
    l0 = jnp.zeros((QT, 1), jnp.float32)
    a0 = jnp.zeros((QT, HD), jnp.float32)
    m, l, acc = lax.fori_loop(0, t + 1, step, (m0, l0, a0))
    o_ref[...] = (acc / l).astype(jnp.bfloat16)


def _attn_call(qkv):
    return pl.pallas_call(
        _attn_body,
        grid=(H, S // QT),
        in_specs=[
            pl.BlockSpec((QT, HD), lambda h, t: (t, h)),
            pl.BlockSpec((S, HD), lambda h, t: (0, H + h // (H // KVH))),
            pl.BlockSpec((S, HD), lambda h, t: (0, H + KVH + h // (H // KVH))),
        ],
        out_specs=pl.BlockSpec((QT, HD), lambda h, t: (t, h)),
        out_shape=jax.ShapeDtypeStruct((S, H * HD), jnp.bfloat16),
        compiler_params=pltpu.CompilerParams(
            dimension_semantics=("arbitrary", "arbitrary")),
    )(qkv, qkv, qkv)


def _wor_body(o_ref, w_ref, r_ref, ln2_ref, wp_ref, x_ref, h2_ref, cw_ref,
              ii_ref):
    xo = r_ref[...] + jnp.dot(o_ref[...], w_ref[...].astype(jnp.bfloat16),
                              preferred_element_type=jnp.float32)
    x_ref[...] = xo
    h = _rms(xo, ln2_ref[...])
    h2_ref[...] = h
    logits = jnp.dot(h, wp_ref[...], preferred_element_type=jnp.float32)
    col = lax.broadcasted_iota(jnp.int32, (SB, 128), 1)
    valid = col < E
    lm = jnp.where(valid, logits, -1e30)
    m = jnp.max(lm, axis=-1, keepdims=True)
    p = jnp.exp(lm - m)
    probs = p / jnp.sum(p, axis=-1, keepdims=True)
    m1 = jnp.max(probs, axis=-1, keepdims=True)
    i1 = jnp.min(jnp.where(probs == m1, col, 127), axis=-1, keepdims=True)
    probs2 = jnp.where(col == i1, -1.0, probs)
    m2 = jnp.max(probs2, axis=-1, keepdims=True)
    i2 = jnp.min(jnp.where(probs2 == m2, col, 127), axis=-1, keepdims=True)
    wsum = m1 + m2
    cw_ref[...] = (jnp.where(col == i1, m1 / wsum, 0.0)
                   + jnp.where(col == i2, m2 / wsum, 0.0))
    ii_ref[...] = jnp.where(col == 0, i1, jnp.where(col == 1, i2, 0))


def _wor_call(o, wo, res, ln2, wp):
    return pl.pallas_call(
        _wor_body,
        grid=(S // SB,),
        in_specs=[
            pl.BlockSpec((SB, H * HD), lambda s: (s, 0)),
            pl.BlockSpec((H * HD, D), lambda s: (0, 0)),
            pl.BlockSpec((SB, D), lambda s: (s, 0)),
            pl.BlockSpec((1, D), lambda s: (0, 0)),
            pl.BlockSpec((D, 128), lambda s: (0, 0)),
        ],
        out_specs=[
            pl.BlockSpec((SB, D), lambda s: (s, 0)),
            pl.BlockSpec((SB, D), lambda s: (s, 0)),
            pl.BlockSpec((SB, 128), lambda s: (s, 0)),
            pl.BlockSpec((SB, 128), lambda s: (s, 0)),
        ],
        out_shape=[
            jax.ShapeDtypeStruct((S, D), jnp.float32),
            jax.ShapeDtypeStruct((S, D), jnp.float32),
            jax.ShapeDtypeStruct((S, 128), jnp.float32),
            jax.ShapeDtypeStruct((S, 128), jnp.int32),
        ],
        compiler_params=pltpu.CompilerParams(
            dimension_semantics=("arbitrary",)),
    )(o, wo, res, ln2, wp)


NW_SC = 32
ROWS_W = (S * TOPK) // NW_SC
CH = 16


def _sc_gather(table, idx):
    mesh = plsc.VectorSubcoreMesh(core_axis_name="c", subcore_axis_name="s")

    nch = ROWS_W // CH

    @functools.partial(
        pl.kernel, mesh=mesh,
        out_type=jax.ShapeDtypeStruct((S * TOPK, D), jnp.float32),
        scratch_types=[
            pltpu.VMEM((ROWS_W,), jnp.int32),
            pltpu.VMEM((CH, D), jnp.float32),
            pltpu.VMEM((CH, D), jnp.float32),
            pltpu.SemaphoreType.DMA,
            pltpu.SemaphoreType.DMA,
        ],
    )
    def k(table_hbm, idx_hbm, out_hbm, idx_v, rows0, rows1, sem0, sem1):
        wid = lax.axis_index("s") * 2 + lax.axis_index("c")
        base = wid * ROWS_W
        pltpu.sync_copy(idx_hbm.at[pl.ds(base, ROWS_W)], idx_v)
        bufs = (rows0, rows1)
        sems = (sem0, sem1)
        handles = [None] * nch
        for c in range(2):
            handles[c] = pltpu.async_copy(
                table_hbm.at[idx_v.at[pl.ds(c * CH, CH)]], bufs[c], sems[c])
        for c in range(nch):
            handles[c].wait()
            pltpu.sync_copy(bufs[c % 2], out_hbm.at[pl.ds(base + c * CH, CH)])
            if c + 2 < nch:
                handles[c + 2] = pltpu.async_copy(
                    table_hbm.at[idx_v.at[pl.ds((c + 2) * CH, CH)]],
                    bufs[c % 2], sems[c % 2])

    return k(table, idx)


def _grouped_body(tile_ref, eid_ref, s0_ref, s1_ref, first_ref,
                  hs_ref, wg_ref, wu_ref, wd_ref, ws_ref, o_ref):
    s = pl.program_id(0)
    x = hs_ref[...].astype(jnp.bfloat16)
    wg = wg_ref[...].reshape(D, F).astype(jnp.bfloat16)
    wu = wu_ref[...].reshape(D, F).astype(jnp.bfloat16)
    wd = wd_ref[...].reshape(F, D).astype(jnp.bfloat16)
    g = jnp.dot(x, wg, preferred_element_type=jnp.float32)
    u = jnp.dot(x, wu, preferred_element_type=jnp.float32)
    y = jnp.dot((g * lax.logistic(g) * u).astype(jnp.bfloat16), wd,
                preferred_element_type=jnp.float32)
    rows = lax.broadcasted_iota(jnp.int32, (TR, 1), 0)
    mask = (rows >= s0_ref[s]) & (rows < s1_ref[s])
    contrib = jnp.where(mask, ws_ref[...] * y, 0.0)

    @pl.when(first_ref[s] == 1)
    def _():
        o_ref[...] = contrib

    @pl.when(first_ref[s] == 0)
    def _():
        o_ref[...] += contrib


def _grouped_call(tile_id, eid, s0, s1, first, hs, wg, wu, wd, ws_col):
    return pl.pallas_call(
        _grouped_body,
        grid_spec=pltpu.PrefetchScalarGridSpec(
            num_scalar_prefetch=5,
            grid=(NSEG,),
            in_specs=[
                pl.BlockSpec((TR, D), lambda s, t, e, a, b, f: (t[s], 0)),
                pl.BlockSpec((1, D, F), lambda s, t, e, a, b, f: (e[s], 0, 0)),
                pl.BlockSpec((1, D, F), lambda s, t, e, a, b, f: (e[s], 0, 0)),
                pl.BlockSpec((1, F, D), lambda s, t, e, a, b, f: (e[s], 0, 0)),
                pl.BlockSpec((TR, 1), lambda s, t, e, a, b, f: (t[s], 0)),
            ],
            out_specs=pl.BlockSpec((TR, D), lambda s, t, e, a, b, f: (t[s], 0)),
        ),
        out_shape=jax.ShapeDtypeStruct((S * TOPK, D), jnp.float32),
    )(tile_id, eid, s0, s1, first, hs, wg, wu, wd, ws_col)


def _fin_body(x_ref, ue_ref, uo_ref, o_ref):
    o_ref[...] = x_ref[...] + ue_ref[...] + uo_ref[...]


def _fin_call(xo, u2):
    return pl.pallas_call(
        _fin_body,
        grid=(S // SB,),
        in_specs=[
            pl.BlockSpec((SB, D), lambda s: (s, 0)),
            pl.BlockSpec((SB, D), lambda s: (s, 0)),
            pl.BlockSpec((SB, D), lambda s: (s, 1)),
        ],
        out_specs=pl.BlockSpec((SB, D), lambda s: (s, 0)),
        out_shape=jax.ShapeDtypeStruct((S, D), jnp.float32),
        compiler_params=pltpu.CompilerParams(
            dimension_semantics=("parallel",)),
    )(xo, u2, u2)


def kernel(hidden_states, ln1_w, Wq, Wk, Wv, q_norm_w, k_norm_w, Wo, ln2_w,
           router_W, W_gate, W_up, W_down):
    x2 = hidden_states.reshape(S, D)
    qkv = _qkv_call(x2, ln1_w.reshape(1, D), q_norm_w.reshape(1, HD),
                    k_norm_w.reshape(1, HD), Wq, Wk, Wv)
    o = _attn_call(qkv)
    wp = jnp.pad(router_W, ((0, 0), (0, 128 - E)))
    xo, h2, cw, ii = _wor_call(o, Wo, x2, ln2_w.reshape(1, D), wp)

    topi = ii[:, :TOPK]
    w_flat = jnp.take_along_axis(cw[:, :E], topi, axis=1).reshape(-1)
    e_flat = topi.reshape(-1)
    order = jnp.argsort(e_flat, stable=True).astype(jnp.int32)
    tok = (order // TOPK).astype(jnp.int32)
    ws_col = w_flat[order][:, None]
    counts = jnp.sum(e_flat[None, :] == jnp.arange(E)[:, None], axis=1)
    csum = jnp.cumsum(counts)
    tile_starts = jnp.arange(NT + 1, dtype=jnp.int32) * TR
    bounds = jnp.sort(jnp.concatenate(
        [tile_starts, csum[:E - 1].astype(jnp.int32)]))
    seg0, seg1 = bounds[:NSEG], bounds[1:]
    tile_id = jnp.clip(seg0 // TR, 0, NT - 1).astype(jnp.int32)
    eid = jnp.clip(jnp.searchsorted(csum, seg0, side='right'),
                   0, E - 1).astype(jnp.int32)
    s0 = (seg0 - tile_id * TR).astype(jnp.int32)
    s1 = (seg1 - tile_id * TR).astype(jnp.int32)
    first = jnp.concatenate([jnp.ones((1,), jnp.int32),
                             (tile_id[1:] != tile_id[:-1]).astype(jnp.int32)])

    hs = _sc_gather(h2, tok)
    os_ = _grouped_call(tile_id, eid, s0, s1, first, hs,
                        W_gate, W_up, W_down, ws_col)
    inv = jnp.argsort(order).astype(jnp.int32)
    u = _sc_gather(os_, inv)
    y = _fin_call(xo, u.reshape(S, 2 * D))
    return y.reshape(1, S, D)
